# baseline (speedup 1.0000x reference)
import functools

import jax
import jax.numpy as jnp
from jax import lax
from jax.experimental import pallas as pl
from jax.experimental.pallas import tpu as pltpu
from jax.experimental.pallas import tpu_sc as plsc

N_NODES = 100000
N_EDGES = 6400000
NC = 2
NS = 16
NW = NC * NS
WW = 2048
G_TOTAL = N_EDGES // WW
NWIN = 98
STRIPE = 6256
ACC_PAD = STRIPE * NS


def _make_sc_kernel():
    mesh = plsc.VectorSubcoreMesh(core_axis_name="c", subcore_axis_name="s")

    @functools.partial(
        pl.kernel,
        mesh=mesh,
        out_type=jax.ShapeDtypeStruct((NC, ACC_PAD), jnp.float32),
        scratch_types=[
            pltpu.VMEM_SHARED((ACC_PAD,), jnp.float32),
            pltpu.VMEM_SHARED((N_NODES,), jnp.float32),
            pltpu.VMEM((2, WW), jnp.int32),
            pltpu.VMEM((2, WW), jnp.int32),
            pltpu.VMEM((WW,), jnp.int32),
            pltpu.VMEM((WW,), jnp.int32),
            pltpu.VMEM((WW,), jnp.int32),
            pltpu.VMEM((WW,), jnp.int32),
            pltpu.VMEM((WW,), jnp.float32),
            pltpu.VMEM((WW,), jnp.float32),
            pltpu.VMEM((WW,), jnp.float32),
            pltpu.VMEM((WW,), jnp.float32),
            pltpu.VMEM((WW,), jnp.float32),
            pltpu.VMEM((WW,), jnp.float32),
            pltpu.SemaphoreType.DMA((2,)),
            pltpu.SemaphoreType.DMA((2,)),
            pltpu.SemaphoreType.DMA((2,)),
            pltpu.SemaphoreType.DMA((2,)),
        ],
    )
    def sc_scatter(rates_hbm, ei_hbm, w_hbm, out_hbm,
                   acc_sh, rates_sh, ed0, ed1, si0, si1, di0, di1,
                   w0, w1, va0, va1, ms0, ms1,
                   sem_ed, sem_w, sem_g, sem_sc):
        c = lax.axis_index("c")
        s = lax.axis_index("s")
        wid = c * NS + s

        ED = [ed0, ed1]
        SI = [si0, si1]
        DI = [di0, di1]
        WB = [w0, w1]
        VA = [va0, va1]
        MS = [ms0, ms1]

        def off_of(k):
            g = wid + NW * k
            g = jnp.where(g < G_TOTAL, g, 0)
            return g * WW

        def issue_in(k, b):
            off = off_of(k)
            pltpu.async_copy(ei_hbm.at[:, pl.ds(off, WW)], ED[b],
                             sem_ed.at[b])
            pltpu.async_copy(w_hbm.at[pl.ds(off, WW)], WB[b], sem_w.at[b])

        def wait_in(k, b):
            off = off_of(k)
            pltpu.make_async_copy(ei_hbm.at[:, pl.ds(off, WW)], ED[b],
                                  sem_ed.at[b]).wait()
            pltpu.make_async_copy(w_hbm.at[pl.ds(off, WW)], WB[b],
                                  sem_w.at[b]).wait()

        def copy_src(b):
            def body(i, mc):
                sl = pl.ds(i * 16, 16)
                SI[b][sl] = ED[b][0, sl]
                return mc
            lax.fori_loop(0, WW // 16, body, 0)

        def issue_gather(b):
            pltpu.async_copy(rates_sh.at[SI[b]], VA[b], sem_g.at[b])

        def wait_gather(b):
            pltpu.make_async_copy(rates_sh.at[SI[b]], VA[b],
                                  sem_g.at[b]).wait()

        def issue_scatter(b):
            pltpu.async_copy(MS[b], acc_sh.at[DI[b]], sem_sc.at[b],
                             add=True)

        def wait_scatter(b):
            pltpu.make_async_copy(MS[b], acc_sh.at[DI[b]],
                                  sem_sc.at[b]).wait()

        def compute(b):
            def mul_body(i, mc):
                sl = pl.ds(i * 16, 16)
                MS[b][sl] = VA[b][sl] * WB[b][sl]
                DI[b][sl] = ED[b][1, sl]
                return mc
            lax.fori_loop(0, WW // 16, mul_body, 0)

        issue_in(0, 0)
        issue_in(1, 1)

        def zero_body(i, carry):
            ms0[pl.ds(i * 16, 16)] = jnp.zeros((16,), jnp.float32)
            return carry
        lax.fori_loop(0, WW // 16, zero_body, 0)
        zoff = s * STRIPE
        pltpu.sync_copy(ms0.at[pl.ds(0, WW)], acc_sh.at[pl.ds(zoff, WW)])
        pltpu.sync_copy(ms0.at[pl.ds(0, WW)], acc_sh.at[pl.ds(zoff + WW, WW)])
        pltpu.sync_copy(ms0.at[pl.ds(0, WW)],
                        acc_sh.at[pl.ds(zoff + 2 * WW, WW)])
        pltpu.sync_copy(ms0.at[pl.ds(0, STRIPE - 3 * WW)],
                        acc_sh.at[pl.ds(zoff + 3 * WW, STRIPE - 3 * WW)])

        @pl.when(s == 0)
        def _():
            pltpu.sync_copy(rates_hbm, rates_sh)

        plsc.subcore_barrier()

        wait_in(0, 0)
        copy_src(0)
        issue_gather(0)

        def outer(t, carry):
            for j in range(2):
                k = 2 * t + j
                b = j
                nb = 1 - j
                wait_in(k + 1, nb)
                copy_src(nb)
                issue_gather(nb)
                wait_gather(b)
                compute(b)
                issue_scatter(b)
                if j == 0:

                    @pl.when(t > 0)
                    def _():
                        wait_scatter(nb)
                else:
                    wait_scatter(nb)
                issue_in(k + 2, b)
            return carry
        lax.fori_loop(0, (NWIN - 2) // 2, outer, 0)

        valid97 = wid + NW * (NWIN - 1) < G_TOTAL
        wait_in(97, 1)
        copy_src(1)
        issue_gather(1)
        wait_gather(0)
        compute(0)
        issue_scatter(0)
        wait_scatter(1)
        wait_gather(1)
        compute(1)

        @pl.when(valid97)
        def _():
            issue_scatter(1)
        wait_scatter(0)

        @pl.when(valid97)
        def _():
            wait_scatter(1)

        plsc.subcore_barrier()

        @pl.when(s == 0)
        def _():
            pltpu.sync_copy(acc_sh, out_hbm.at[c])

    return sc_scatter


_sc_scatter = _make_sc_kernel()

_ROWS = 8
_COLS = N_NODES // _ROWS


def _epilogue_body(p0_ref, p1_ref, rates_ref, gain_ref, tau_ref, base_ref,
                   out_ref):
    syn = p0_ref[...] + p1_ref[...]
    act = jnp.tanh(syn + base_ref[...])
    out_ref[...] = (1.0 / tau_ref[...]) * (gain_ref[...] * act - rates_ref[...])


def kernel(rates, gain, time_constant, baseline, edge_weight, edge_index):
    ei = edge_index.astype(jnp.int32)
    partials = _sc_scatter(rates, ei, edge_weight)
    p0 = partials[0, :N_NODES].reshape(_ROWS, _COLS)
    p1 = partials[1, :N_NODES].reshape(_ROWS, _COLS)
    out = pl.pallas_call(
        _epilogue_body,
        out_shape=jax.ShapeDtypeStruct((_ROWS, _COLS), jnp.float32),
    )(p0, p1,
      rates.reshape(_ROWS, _COLS),
      gain.reshape(_ROWS, _COLS),
      time_constant.reshape(_ROWS, _COLS),
      baseline.reshape(_ROWS, _COLS))
    return out.reshape(N_NODES)


# speedup vs baseline: 1.4257x; 1.4257x over previous
import functools

import jax
import jax.numpy as jnp
from jax import lax
from jax.experimental import pallas as pl
from jax.experimental.pallas import tpu as pltpu
from jax.experimental.pallas import tpu_sc as plsc

N_NODES = 100000
N_EDGES = 6400000
NC = 2
NS = 16
NW = NC * NS
EDGES_PER_WORKER = N_EDGES // NW
W = 2000
NWIN = EDGES_PER_WORKER // W
NB2 = 2
NB4 = 4
STRIPE = 6256
ACC_PAD = STRIPE * NS


def _make_sc_kernel():
    mesh = plsc.VectorSubcoreMesh(core_axis_name="c", subcore_axis_name="s")

    @functools.partial(
        pl.kernel,
        mesh=mesh,
        out_type=jax.ShapeDtypeStruct((NC, ACC_PAD), jnp.float32),
        scratch_types=[
            pltpu.VMEM_SHARED((ACC_PAD,), jnp.float32),
            pltpu.VMEM_SHARED((N_NODES,), jnp.float32),
            pltpu.VMEM((NB2 * W,), jnp.int32),
            pltpu.VMEM((NB4 * W,), jnp.int32),
            pltpu.VMEM((NB2 * W,), jnp.float32),
            pltpu.VMEM((NB2 * W,), jnp.float32),
            pltpu.VMEM((NB4 * W,), jnp.float32),
            pltpu.SemaphoreType.DMA((NB2,)),
            pltpu.SemaphoreType.DMA((NB2,)),
            pltpu.SemaphoreType.DMA((NB2,)),
            pltpu.SemaphoreType.DMA((NB4,)),
        ],
    )
    def sc_scatter(rates_hbm, ei_hbm, w_hbm, out_hbm,
                   acc_sh, rates_sh, src_v, dst_v, w_v, vals_v, msg_v,
                   sem_src, sem_wd, sem_g, sem_sc):
        c = lax.axis_index("c")
        s = lax.axis_index("s")
        wid = c * NS + s
        base = wid * EDGES_PER_WORKER

        def issue_in(k, b2, b4):
            off = base + k * W
            pltpu.async_copy(ei_hbm.at[pl.ds(off, W)],
                             src_v.at[pl.ds(b2 * W, W)], sem_src.at[b2])
            pltpu.async_copy(w_hbm.at[pl.ds(off, W)],
                             w_v.at[pl.ds(b2 * W, W)], sem_wd.at[b2])
            pltpu.async_copy(ei_hbm.at[pl.ds(N_EDGES + off, W)],
                             dst_v.at[pl.ds(b4 * W, W)], sem_wd.at[b2])

        def wait_src(k, b2):
            off = base + k * W
            pltpu.make_async_copy(ei_hbm.at[pl.ds(off, W)],
                                  src_v.at[pl.ds(b2 * W, W)],
                                  sem_src.at[b2]).wait()

        def wait_wd(k, b2, b4):
            off = base + k * W
            pltpu.make_async_copy(w_hbm.at[pl.ds(off, W)],
                                  w_v.at[pl.ds(b2 * W, W)],
                                  sem_wd.at[b2]).wait()
            pltpu.make_async_copy(ei_hbm.at[pl.ds(N_EDGES + off, W)],
                                  dst_v.at[pl.ds(b4 * W, W)],
                                  sem_wd.at[b2]).wait()

        def issue_gather(b2):
            sl = pl.ds(b2 * W, W)
            pltpu.async_copy(rates_sh.at[src_v.at[sl]], vals_v.at[sl],
                             sem_g.at[b2])

        def wait_gather(b2):
            sl = pl.ds(b2 * W, W)
            pltpu.make_async_copy(rates_sh.at[src_v.at[sl]], vals_v.at[sl],
                                  sem_g.at[b2]).wait()

        def issue_scatter(b4):
            sl = pl.ds(b4 * W, W)
            pltpu.async_copy(msg_v.at[sl], acc_sh.at[dst_v.at[sl]],
                             sem_sc.at[b4], add=True)

        def wait_scatter(b4):
            sl = pl.ds(b4 * W, W)
            pltpu.make_async_copy(msg_v.at[sl], acc_sh.at[dst_v.at[sl]],
                                  sem_sc.at[b4]).wait()

        def compute(b2, b4):
            def mul_body(i, mc):
                s2 = pl.ds(b2 * W + i * 16, 16)
                s4 = pl.ds(b4 * W + i * 16, 16)
                msg_v[s4] = vals_v[s2] * w_v[s2]
                return mc
            lax.fori_loop(0, W // 16, mul_body, 0)

        issue_in(0, 0, 0)
        issue_in(1, 1, 1)

        def zero_body(i, carry):
            msg_v[pl.ds(i * 16, 16)] = jnp.zeros((16,), jnp.float32)
            return carry
        lax.fori_loop(0, W // 16, zero_body, 0)
        zoff = s * STRIPE
        pltpu.sync_copy(msg_v.at[pl.ds(0, W)], acc_sh.at[pl.ds(zoff, W)])
        pltpu.sync_copy(msg_v.at[pl.ds(0, W)], acc_sh.at[pl.ds(zoff + W, W)])
        pltpu.sync_copy(msg_v.at[pl.ds(0, W)],
                        acc_sh.at[pl.ds(zoff + 2 * W, W)])
        pltpu.sync_copy(msg_v.at[pl.ds(0, STRIPE - 3 * W)],
                        acc_sh.at[pl.ds(zoff + 3 * W, STRIPE - 3 * W)])

        @pl.when(s == 0)
        def _():
            pltpu.sync_copy(rates_hbm, rates_sh)

        plsc.subcore_barrier()

        wait_src(0, 0)
        issue_gather(0)

        def outer(t, carry):
            for j in range(4):
                k = 4 * t + j
                b2 = j % NB2
                b4 = j

                def adv():
                    wait_src(k + 1, (j + 1) % NB2)
                    issue_gather((j + 1) % NB2)
                if j == 3:
                    @pl.when(t < NWIN // 4 - 1)
                    def _():
                        adv()
                else:
                    adv()

                if j >= 2:
                    wait_scatter(j - 2)
                else:
                    @pl.when(t > 0)
                    def _():
                        wait_scatter(j + 2)

                wait_gather(b2)
                wait_wd(k, b2, b4)
                compute(b2, b4)
                issue_scatter(b4)

                if j >= 2:
                    @pl.when(t < NWIN // 4 - 1)
                    def _():
                        issue_in(k + 2, b2, (j + 2) % NB4)
                else:
                    issue_in(k + 2, b2, (j + 2) % NB4)
            return carry
        lax.fori_loop(0, NWIN // 4, outer, 0)

        wait_scatter(2)
        wait_scatter(3)

        plsc.subcore_barrier()

        @pl.when(s == 0)
        def _():
            pltpu.sync_copy(acc_sh, out_hbm.at[c])

    return sc_scatter


_sc_scatter = _make_sc_kernel()

_ROWS = 8
_COLS = N_NODES // _ROWS


def _epilogue_body(p0_ref, p1_ref, rates_ref, gain_ref, tau_ref, base_ref,
                   out_ref):
    syn = p0_ref[...] + p1_ref[...]
    act = jnp.tanh(syn + base_ref[...])
    out_ref[...] = (1.0 / tau_ref[...]) * (gain_ref[...] * act - rates_ref[...])


def kernel(rates, gain, time_constant, baseline, edge_weight, edge_index):
    ei = edge_index.astype(jnp.int32).reshape(2 * N_EDGES)
    partials = _sc_scatter(rates, ei, edge_weight)
    p0 = partials[0, :N_NODES].reshape(_ROWS, _COLS)
    p1 = partials[1, :N_NODES].reshape(_ROWS, _COLS)
    out = pl.pallas_call(
        _epilogue_body,
        out_shape=jax.ShapeDtypeStruct((_ROWS, _COLS), jnp.float32),
    )(p0, p1,
      rates.reshape(_ROWS, _COLS),
      gain.reshape(_ROWS, _COLS),
      time_constant.reshape(_ROWS, _COLS),
      baseline.reshape(_ROWS, _COLS))
    return out.reshape(N_NODES)
